# baseline (speedup 1.0000x reference)
import functools

import jax
import jax.numpy as jnp
from jax import lax
from jax.experimental import pallas as pl
from jax.experimental.pallas import tpu as pltpu
from jax.experimental.pallas import tpu_sc as plsc

_NC = 2
_NS = 16
_NW = _NC * _NS
_LANES = 16

_TAB_PAD = 1024


def _sc_gather_body(sa_tab_hbm, som_tab_hbm, t_hbm, sa_out_hbm, som_out_hbm,
                    t_v, sa_o_v, som_o_v, sem_a, sem_b, *, b_per_w):
    wid = lax.axis_index("s") * _NC + lax.axis_index("c")
    base = wid * b_per_w
    pltpu.sync_copy(t_hbm.at[pl.ds(base, b_per_w)], t_v)
    cp_a = pltpu.async_copy(sa_tab_hbm.at[t_v], sa_o_v, sem_a)
    cp_b = pltpu.async_copy(som_tab_hbm.at[t_v], som_o_v, sem_b)
    cp_a.wait()
    cp_b.wait()
    pltpu.sync_copy(sa_o_v, sa_out_hbm.at[pl.ds(base, b_per_w)])
    pltpu.sync_copy(som_o_v, som_out_hbm.at[pl.ds(base, b_per_w)])


def _sc_gather(sa_tab, som_tab, t):
    b = t.shape[0]
    b_per_w = b // _NW
    mesh = plsc.VectorSubcoreMesh(core_axis_name="c", subcore_axis_name="s")
    body = functools.partial(_sc_gather_body, b_per_w=b_per_w)
    k = pl.kernel(
        body,
        out_type=(
            jax.ShapeDtypeStruct((b,), jnp.float32),
            jax.ShapeDtypeStruct((b,), jnp.float32),
        ),
        mesh=mesh,
        scratch_types=[
            pltpu.VMEM((b_per_w,), jnp.int32),
            pltpu.VMEM((b_per_w,), jnp.float32),
            pltpu.VMEM((b_per_w,), jnp.float32),
            pltpu.SemaphoreType.DMA,
            pltpu.SemaphoreType.DMA,
        ],
    )
    return k(sa_tab, som_tab, t)


def _combine_body(sa_ref, som_ref, x_ref, n_ref, o_ref):
    o_ref[...] = sa_ref[...] * x_ref[...] + som_ref[...] * n_ref[...]


def _combine(sa_col, som_col, x2, n2, block_rows):
    b, d = x2.shape
    grid = (b // block_rows,)
    row_spec = pl.BlockSpec((block_rows, d), lambda i: (i, 0))
    coef_spec = pl.BlockSpec((block_rows, 1), lambda i: (i, 0))
    return pl.pallas_call(
        _combine_body,
        grid=grid,
        in_specs=[coef_spec, coef_spec, row_spec, row_spec],
        out_specs=row_spec,
        out_shape=jax.ShapeDtypeStruct((b, d), jnp.float32),
    )(sa_col, som_col, x2, n2)


def kernel(x_0, t, noise, sqrt_alphas_cumprod, sqrt_one_minus_alphas_cumprod):
    b = t.shape[0]
    c, l = x_0.shape[1], x_0.shape[2]
    tt = sqrt_alphas_cumprod.shape[0]
    pad = _TAB_PAD - tt
    sa_tab = jnp.pad(sqrt_alphas_cumprod, (0, pad))
    som_tab = jnp.pad(sqrt_one_minus_alphas_cumprod, (0, pad))

    sa_b, som_b = _sc_gather(sa_tab, som_tab, t)

    x2 = x_0.reshape(b, c * l)
    n2 = noise.reshape(b, c * l)
    out = _combine(sa_b.reshape(b, 1), som_b.reshape(b, 1), x2, n2, 256)
    return out.reshape(b, c, l)


# speedup vs baseline: 2.3992x; 2.3992x over previous
import functools

import jax
import jax.numpy as jnp
from jax import lax
from jax.experimental import pallas as pl
from jax.experimental.pallas import tpu as pltpu
from jax.experimental.pallas import tpu_sc as plsc

_NC = 2
_NS = 16
_NW = _NC * _NS
_LANES = 16

_TAB_PAD = 1024


def _sc_gather_body(sa_tab_hbm, som_tab_hbm, t_hbm, sa_out_hbm, som_out_hbm,
                    t_v, sa_o_v, som_o_v, sem_a, sem_b, *, b_per_w):
    wid = lax.axis_index("s") * _NC + lax.axis_index("c")
    base = wid * b_per_w
    pltpu.sync_copy(t_hbm.at[pl.ds(base, b_per_w)], t_v)
    cp_a = pltpu.async_copy(sa_tab_hbm.at[t_v], sa_o_v, sem_a)
    cp_b = pltpu.async_copy(som_tab_hbm.at[t_v], som_o_v, sem_b)
    cp_a.wait()
    cp_b.wait()
    pltpu.sync_copy(sa_o_v, sa_out_hbm.at[pl.ds(base, b_per_w)])
    pltpu.sync_copy(som_o_v, som_out_hbm.at[pl.ds(base, b_per_w)])


def _sc_gather(sa_tab, som_tab, t):
    b = t.shape[0]
    b_per_w = b // _NW
    mesh = plsc.VectorSubcoreMesh(core_axis_name="c", subcore_axis_name="s")
    body = functools.partial(_sc_gather_body, b_per_w=b_per_w)
    k = pl.kernel(
        body,
        out_type=(
            jax.ShapeDtypeStruct((b,), jnp.float32),
            jax.ShapeDtypeStruct((b,), jnp.float32),
        ),
        mesh=mesh,
        scratch_types=[
            pltpu.VMEM((b_per_w,), jnp.int32),
            pltpu.VMEM((b_per_w,), jnp.float32),
            pltpu.VMEM((b_per_w,), jnp.float32),
            pltpu.SemaphoreType.DMA,
            pltpu.SemaphoreType.DMA,
        ],
    )
    return k(sa_tab, som_tab, t)


def _combine_body(sa_ref, som_ref, x_ref, n_ref, o_ref):
    o_ref[...] = sa_ref[...] * x_ref[...] + som_ref[...] * n_ref[...]


def _combine(sa_col, som_col, x, n, block_rows):
    b, c, l = x.shape
    grid = (b // block_rows,)
    row_spec = pl.BlockSpec((block_rows, c, l), lambda i: (i, 0, 0))
    coef_spec = pl.BlockSpec((block_rows, 1, 1), lambda i: (i, 0, 0))
    return pl.pallas_call(
        _combine_body,
        grid=grid,
        in_specs=[coef_spec, coef_spec, row_spec, row_spec],
        out_specs=row_spec,
        out_shape=jax.ShapeDtypeStruct((b, c, l), jnp.float32),
    )(sa_col, som_col, x, n)


def kernel(x_0, t, noise, sqrt_alphas_cumprod, sqrt_one_minus_alphas_cumprod):
    b = t.shape[0]
    c, l = x_0.shape[1], x_0.shape[2]
    tt = sqrt_alphas_cumprod.shape[0]
    pad = _TAB_PAD - tt
    sa_tab = jnp.pad(sqrt_alphas_cumprod, (0, pad))
    som_tab = jnp.pad(sqrt_one_minus_alphas_cumprod, (0, pad))

    sa_b, som_b = _sc_gather(sa_tab, som_tab, t)

    return _combine(sa_b.reshape(b, 1, 1), som_b.reshape(b, 1, 1),
                    x_0, noise, 256)


# speedup vs baseline: 2.4240x; 1.0103x over previous
import functools

import jax
import jax.numpy as jnp
from jax import lax
from jax.experimental import pallas as pl
from jax.experimental.pallas import tpu as pltpu
from jax.experimental.pallas import tpu_sc as plsc

_NC = 2
_NS = 16
_NW = _NC * _NS
_LANES = 16

_TAB_PAD = 1024


def _sc_gather_body(sa_tab_hbm, som_tab_hbm, t_hbm, sa_out_hbm, som_out_hbm,
                    t_v, sa_o_v, som_o_v, sem_a, sem_b, *, b_per_w):
    wid = lax.axis_index("s") * _NC + lax.axis_index("c")
    base = wid * b_per_w
    pltpu.sync_copy(t_hbm.at[pl.ds(base, b_per_w)], t_v)
    cp_a = pltpu.async_copy(sa_tab_hbm.at[t_v], sa_o_v, sem_a)
    cp_b = pltpu.async_copy(som_tab_hbm.at[t_v], som_o_v, sem_b)
    cp_a.wait()
    cp_b.wait()
    pltpu.sync_copy(sa_o_v, sa_out_hbm.at[pl.ds(base, b_per_w)])
    pltpu.sync_copy(som_o_v, som_out_hbm.at[pl.ds(base, b_per_w)])


def _sc_gather(sa_tab, som_tab, t):
    b = t.shape[0]
    b_per_w = b // _NW
    mesh = plsc.VectorSubcoreMesh(core_axis_name="c", subcore_axis_name="s")
    body = functools.partial(_sc_gather_body, b_per_w=b_per_w)
    k = pl.kernel(
        body,
        out_type=(
            jax.ShapeDtypeStruct((b,), jnp.float32),
            jax.ShapeDtypeStruct((b,), jnp.float32),
        ),
        mesh=mesh,
        scratch_types=[
            pltpu.VMEM((b_per_w,), jnp.int32),
            pltpu.VMEM((b_per_w,), jnp.float32),
            pltpu.VMEM((b_per_w,), jnp.float32),
            pltpu.SemaphoreType.DMA,
            pltpu.SemaphoreType.DMA,
        ],
    )
    return k(sa_tab, som_tab, t)


def _combine_body(sa_ref, som_ref, x_ref, n_ref, o_ref):
    o_ref[...] = sa_ref[...] * x_ref[...] + som_ref[...] * n_ref[...]


def _combine(sa_col, som_col, x, n, block_rows):
    b, c, l = x.shape
    grid = (b // block_rows,)
    row_spec = pl.BlockSpec((block_rows, c, l), lambda i: (i, 0, 0))
    coef_spec = pl.BlockSpec((block_rows, 1, 1), lambda i: (i, 0, 0))
    return pl.pallas_call(
        _combine_body,
        grid=grid,
        in_specs=[coef_spec, coef_spec, row_spec, row_spec],
        out_specs=row_spec,
        out_shape=jax.ShapeDtypeStruct((b, c, l), jnp.float32),
    )(sa_col, som_col, x, n)


def kernel(x_0, t, noise, sqrt_alphas_cumprod, sqrt_one_minus_alphas_cumprod):
    b = t.shape[0]
    c, l = x_0.shape[1], x_0.shape[2]
    tt = sqrt_alphas_cumprod.shape[0]
    pad = _TAB_PAD - tt
    sa_tab = jnp.pad(sqrt_alphas_cumprod, (0, pad))
    som_tab = jnp.pad(sqrt_one_minus_alphas_cumprod, (0, pad))

    sa_b, som_b = _sc_gather(sa_tab, som_tab, t)

    return _combine(sa_b.reshape(b, 1, 1), som_b.reshape(b, 1, 1),
                    x_0, noise, 512)
